# baseline (speedup 1.0000x reference)
import jax
import jax.numpy as jnp
from jax import lax
from jax.experimental import pallas as pl
from jax.experimental.pallas import tpu as pltpu
from jax.experimental.pallas import tpu_sc as plsc

_B = 16384
_C = 1000
_TC_ROWS = 8192
_SC_ROWS = _B - _TC_ROWS
_BLK = 512

_NC, _NS, _L = 2, 16, 16
_NW = _NC * _NS
_ROWS_PER_TILE = _SC_ROWS // _NW
_CHUNK = 32
_NCHUNK = _ROWS_PER_TILE // _CHUNK
_GROUPS = _CHUNK // _L
_UNROLL = 8
_STEPS = _C // _UNROLL


def _tc_body(x_ref, logit_ref, ema_ref, out_ref):
    x = x_ref[...]
    blk, c = x.shape
    m = jnp.max(x, axis=1, keepdims=True)
    iota_f = jax.lax.broadcasted_iota(jnp.int32, (blk, c), 1).astype(jnp.float32)
    idxf = jnp.min(jnp.where(x == m, iota_f, 1024.0), axis=1, keepdims=True)
    ema_b = jnp.broadcast_to(ema_ref[...], (blk, c))
    thr = jnp.max(jnp.where(iota_f == idxf, ema_b, -jnp.inf), axis=1,
                  keepdims=True)
    diff = jnp.maximum(thr - logit_ref[...], 0.0)
    out_ref[...] = diff * diff


def _tc_part(gen_logits, gen_classes, ema_real):
    c = _C
    return pl.pallas_call(
        _tc_body,
        grid=(_TC_ROWS // _BLK,),
        in_specs=[
            pl.BlockSpec((_BLK, c), lambda i: (i, 0)),
            pl.BlockSpec((_BLK, 1), lambda i: (i, 0)),
            pl.BlockSpec((1, c), lambda i: (0, 0)),
        ],
        out_specs=pl.BlockSpec((_BLK, 1), lambda i: (i, 0)),
        out_shape=jax.ShapeDtypeStruct((_TC_ROWS, 1), jnp.float32),
        compiler_params=pltpu.CompilerParams(
            dimension_semantics=("arbitrary",),
        ),
    )(gen_classes, gen_logits, ema_real.reshape(1, c))


def _combine(aval, aidx, bval, bidx):
    pred = bval > aval
    return jnp.where(pred, bval, aval), jnp.where(pred, bidx, aidx)


def _sc_body(classes_hbm, logits_hbm, ema_hbm, out_hbm,
             xbuf0, xbuf1, ema_v, logit_v, loss_v, sem0, sem1):
    wid = lax.axis_index("s") * _NC + lax.axis_index("c")
    base = _TC_ROWS + wid * _ROWS_PER_TILE
    pltpu.sync_copy(ema_hbm, ema_v)
    pltpu.sync_copy(logits_hbm.at[pl.ds(base, _ROWS_PER_TILE)], logit_v)
    row16 = lax.iota(jnp.int32, _L)

    bufs = (xbuf0, xbuf1)
    sems = (sem0, sem1)

    def chunk_dma(chunk):
        row0 = base + chunk * _CHUNK
        return pltpu.make_async_copy(
            classes_hbm.at[pl.ds(row0 * _C, _CHUNK * _C)],
            bufs[chunk % 2], sems[chunk % 2])

    chunk_dma(0).start()
    uconst = [jnp.full((_L,), u, dtype=jnp.int32) for u in range(_UNROLL)]

    for chunk in range(_NCHUNK):
        xbuf = bufs[chunk % 2]
        chunk_dma(chunk).wait()
        if chunk + 1 < _NCHUNK:
            chunk_dma(chunk + 1).start()
        for g in range(_GROUPS):
            rowbase = (row16 + (g * _L)) * _C

            def body(i, carry, rowbase=rowbase, xbuf=xbuf):
                best, bidx = carry
                j0 = i * _UNROLL
                vs = [plsc.load_gather(xbuf, [rowbase + (j0 + u)])
                      for u in range(_UNROLL)]
                pairs = [(vs[u], uconst[u]) for u in range(_UNROLL)]
                while len(pairs) > 1:
                    nxt = []
                    for k in range(0, len(pairs), 2):
                        nxt.append(_combine(pairs[k][0], pairs[k][1],
                                            pairs[k + 1][0], pairs[k + 1][1]))
                    pairs = nxt
                val, idx = pairs[0]
                jabs = jnp.full((_L,), j0, dtype=jnp.int32) + idx
                best, bidx = _combine(best, bidx, val, jabs)
                return best, bidx

            init = (jnp.full((_L,), -jnp.inf, jnp.float32),
                    jnp.zeros((_L,), jnp.int32))
            best, bidx = lax.fori_loop(0, _STEPS, body, init)
            thr = plsc.load_gather(ema_v, [bidx])
            lg = logit_v[pl.ds(chunk * _CHUNK + g * _L, _L)]
            d = jnp.maximum(thr - lg, 0.0)
            loss_v[pl.ds(chunk * _CHUNK + g * _L, _L)] = d * d
    pltpu.sync_copy(loss_v, out_hbm.at[pl.ds(wid * _ROWS_PER_TILE,
                                             _ROWS_PER_TILE)])


def _sc_part(gen_logits, gen_classes, ema_real):
    b, c = _B, _C
    mesh = plsc.VectorSubcoreMesh(core_axis_name="c", subcore_axis_name="s")
    f = pl.kernel(
        _sc_body,
        out_type=jax.ShapeDtypeStruct((_SC_ROWS,), jnp.float32),
        mesh=mesh,
        compiler_params=pltpu.CompilerParams(
            needs_layout_passes=False,
            use_tc_tiling_on_sc=False,
        ),
        scratch_types=[
            pltpu.VMEM((_CHUNK * c,), jnp.float32),
            pltpu.VMEM((_CHUNK * c,), jnp.float32),
            pltpu.VMEM((c,), jnp.float32),
            pltpu.VMEM((_ROWS_PER_TILE,), jnp.float32),
            pltpu.VMEM((_ROWS_PER_TILE,), jnp.float32),
            pltpu.SemaphoreType.DMA,
            pltpu.SemaphoreType.DMA,
        ],
    )
    return f(gen_classes.reshape(b * c), gen_logits.reshape(b), ema_real)


def kernel(gen_logits, gen_classes, ema_real):
    tc_out = _tc_part(gen_logits, gen_classes, ema_real)
    sc_out = _sc_part(gen_logits, gen_classes, ema_real)
    return jnp.concatenate([tc_out, sc_out.reshape(_SC_ROWS, 1)], axis=0)


# speedup vs baseline: 1.0016x; 1.0016x over previous
import jax
import jax.numpy as jnp
from jax import lax
from jax.experimental import pallas as pl
from jax.experimental.pallas import tpu as pltpu
from jax.experimental.pallas import tpu_sc as plsc

_B = 16384
_C = 1000
_TC_ROWS = 8192
_SC_ROWS = _B - _TC_ROWS
_BLK = 512

_NC, _NS, _L = 2, 16, 16
_NW = _NC * _NS
_ROWS_PER_TILE = _SC_ROWS // _NW
_CHUNK = 32
_NCHUNK = _ROWS_PER_TILE // _CHUNK
_GROUPS = _CHUNK // _L
_UNROLL = 8
_STEPS = _C // _UNROLL


def _tc_body(x_ref, logit_ref, ema_ref, out_ref):
    x = x_ref[...]
    blk, c = x.shape
    m = jnp.max(x, axis=1, keepdims=True)
    iota_f = jax.lax.broadcasted_iota(jnp.int32, (blk, c), 1).astype(jnp.float32)
    idxf = jnp.min(jnp.where(x == m, iota_f, 1024.0), axis=1, keepdims=True)
    ema_b = jnp.broadcast_to(ema_ref[...], (blk, c))
    thr = jnp.max(jnp.where(iota_f == idxf, ema_b, -jnp.inf), axis=1,
                  keepdims=True)
    diff = jnp.maximum(thr - logit_ref[...], 0.0)
    out_ref[...] = diff * diff


def _tc_part(gen_logits, gen_classes, ema_real):
    c = _C
    return pl.pallas_call(
        _tc_body,
        grid=(_TC_ROWS // _BLK,),
        in_specs=[
            pl.BlockSpec((_BLK, c), lambda i: (i, 0)),
            pl.BlockSpec((_BLK, 1), lambda i: (i, 0)),
            pl.BlockSpec((1, c), lambda i: (0, 0)),
        ],
        out_specs=pl.BlockSpec((_BLK, 1), lambda i: (i, 0)),
        out_shape=jax.ShapeDtypeStruct((_TC_ROWS, 1), jnp.float32),
        compiler_params=pltpu.CompilerParams(
            dimension_semantics=("arbitrary",),
        ),
    )(gen_classes, gen_logits, ema_real.reshape(1, c))


def _combine(aval, aidx, bval, bidx):
    pred = bval > aval
    return jnp.where(pred, bval, aval), jnp.where(pred, bidx, aidx)


def _sc_body(classes_hbm, logits_hbm, ema_hbm, out_hbm,
             xbuf0, xbuf1, ema_v, logit_v, loss_v, sem0, sem1):
    wid = lax.axis_index("s") * _NC + lax.axis_index("c")
    base = _TC_ROWS + wid * _ROWS_PER_TILE
    pltpu.sync_copy(ema_hbm, ema_v)
    pltpu.sync_copy(logits_hbm.at[pl.ds(base, _ROWS_PER_TILE)], logit_v)
    row16 = lax.iota(jnp.int32, _L)

    bufs = (xbuf0, xbuf1)
    sems = (sem0, sem1)

    def chunk_dma(chunk):
        row0 = base + chunk * _CHUNK
        return pltpu.make_async_copy(
            classes_hbm.at[pl.ds(row0, _CHUNK), :],
            bufs[chunk % 2], sems[chunk % 2])

    chunk_dma(0).start()
    uconst = [jnp.full((_L,), u, dtype=jnp.int32) for u in range(_UNROLL)]

    for chunk in range(_NCHUNK):
        xbuf = bufs[chunk % 2]
        chunk_dma(chunk).wait()
        if chunk + 1 < _NCHUNK:
            chunk_dma(chunk + 1).start()
        for g in range(_GROUPS):
            rows = row16 + (g * _L)

            def body(i, carry, rows=rows, xbuf=xbuf):
                best, bidx = carry
                j0 = i * _UNROLL
                jsplat = jnp.full((_L,), j0, dtype=jnp.int32)
                vs = [plsc.load_gather(xbuf, [rows, jsplat + uconst[u]])
                      for u in range(_UNROLL)]
                pairs = [(vs[u], uconst[u]) for u in range(_UNROLL)]
                while len(pairs) > 1:
                    nxt = []
                    for k in range(0, len(pairs), 2):
                        nxt.append(_combine(pairs[k][0], pairs[k][1],
                                            pairs[k + 1][0], pairs[k + 1][1]))
                    pairs = nxt
                val, idx = pairs[0]
                jabs = jsplat + idx
                best, bidx = _combine(best, bidx, val, jabs)
                return best, bidx

            init = (jnp.full((_L,), -jnp.inf, jnp.float32),
                    jnp.zeros((_L,), jnp.int32))
            best, bidx = lax.fori_loop(0, _STEPS, body, init)
            thr = plsc.load_gather(ema_v, [bidx])
            lg = logit_v[pl.ds(chunk * _CHUNK + g * _L, _L)]
            d = jnp.maximum(thr - lg, 0.0)
            loss_v[pl.ds(chunk * _CHUNK + g * _L, _L)] = d * d
    pltpu.sync_copy(loss_v, out_hbm.at[pl.ds(wid * _ROWS_PER_TILE,
                                             _ROWS_PER_TILE)])


def _sc_part(gen_logits, gen_classes, ema_real):
    b, c = _B, _C
    mesh = plsc.VectorSubcoreMesh(core_axis_name="c", subcore_axis_name="s")
    f = pl.kernel(
        _sc_body,
        out_type=jax.ShapeDtypeStruct((_SC_ROWS,), jnp.float32),
        mesh=mesh,
        compiler_params=pltpu.CompilerParams(
            needs_layout_passes=False,
            use_tc_tiling_on_sc=False,
        ),
        scratch_types=[
            pltpu.VMEM((_CHUNK, c), jnp.float32),
            pltpu.VMEM((_CHUNK, c), jnp.float32),
            pltpu.VMEM((c,), jnp.float32),
            pltpu.VMEM((_ROWS_PER_TILE,), jnp.float32),
            pltpu.VMEM((_ROWS_PER_TILE,), jnp.float32),
            pltpu.SemaphoreType.DMA,
            pltpu.SemaphoreType.DMA,
        ],
    )
    return f(gen_classes, gen_logits.reshape(b), ema_real)


def kernel(gen_logits, gen_classes, ema_real):
    tc_out = _tc_part(gen_logits, gen_classes, ema_real)
    sc_out = _sc_part(gen_logits, gen_classes, ema_real)
    return jnp.concatenate([tc_out, sc_out.reshape(_SC_ROWS, 1)], axis=0)


# speedup vs baseline: 1.8710x; 1.8681x over previous
import jax
import jax.numpy as jnp
from jax.experimental import pallas as pl
from jax.experimental.pallas import tpu as pltpu

_BLK = 512


def _body(x_ref, logit_ref, ema_ref, out_ref):
    x = x_ref[...]
    blk, c = x.shape
    m = jnp.max(x, axis=1, keepdims=True)
    iota_f = jax.lax.broadcasted_iota(jnp.int32, (blk, c), 1).astype(jnp.float32)
    idxf = jnp.min(jnp.where(x == m, iota_f, 1024.0), axis=1, keepdims=True)
    ema_b = jnp.broadcast_to(ema_ref[...], (blk, c))
    thr = jnp.max(jnp.where(iota_f == idxf, ema_b, -jnp.inf), axis=1,
                  keepdims=True)
    diff = jnp.maximum(thr[:, 0] - logit_ref[...], 0.0)
    out_ref[...] = diff * diff


def kernel(gen_logits, gen_classes, ema_real):
    b, c = gen_classes.shape
    grid = b // _BLK
    out = pl.pallas_call(
        _body,
        grid=(grid,),
        in_specs=[
            pl.BlockSpec((_BLK, c), lambda i: (i, 0)),
            pl.BlockSpec((_BLK,), lambda i: (i,)),
            pl.BlockSpec((1, c), lambda i: (0, 0)),
        ],
        out_specs=pl.BlockSpec((_BLK,), lambda i: (i,)),
        out_shape=jax.ShapeDtypeStruct((b,), jnp.float32),
        compiler_params=pltpu.CompilerParams(
            dimension_semantics=("arbitrary",),
        ),
    )(gen_classes, gen_logits.reshape(b), ema_real.reshape(1, c))
    return out.reshape(b, 1)


# speedup vs baseline: 6.8611x; 3.6670x over previous
import jax
import jax.numpy as jnp
from jax.experimental import pallas as pl
from jax.experimental.pallas import tpu as pltpu

_BN = 2048


def _body(x_ref, logit_ref, ema_ref, out_ref):
    x = x_ref[...]
    c, bn = x.shape
    m = jnp.max(x, axis=0, keepdims=True)
    iota_f = jax.lax.broadcasted_iota(jnp.int32, (c, bn), 0).astype(jnp.float32)
    idxf = jnp.min(jnp.where(x == m, iota_f, 1024.0), axis=0, keepdims=True)
    ema_b = jnp.broadcast_to(ema_ref[...], (c, bn))
    thr = jnp.max(jnp.where(iota_f == idxf, ema_b, -jnp.inf), axis=0,
                  keepdims=True)
    d = jnp.maximum(thr[0] - logit_ref[...], 0.0)
    out_ref[...] = d * d


def kernel(gen_logits, gen_classes, ema_real):
    b, c = gen_classes.shape
    grid = b // _BN
    out = pl.pallas_call(
        _body,
        grid=(grid,),
        in_specs=[
            pl.BlockSpec((c, _BN), lambda i: (0, i)),
            pl.BlockSpec((_BN,), lambda i: (i,)),
            pl.BlockSpec((c, 1), lambda i: (0, 0)),
        ],
        out_specs=pl.BlockSpec((_BN,), lambda i: (i,)),
        out_shape=jax.ShapeDtypeStruct((b,), jnp.float32),
        compiler_params=pltpu.CompilerParams(
            dimension_semantics=("arbitrary",),
        ),
    )(gen_classes.T, gen_logits.reshape(b), ema_real.reshape(c, 1))
    return out.reshape(b, 1)
